# Initial kernel scaffold; baseline (speedup 1.0000x reference)
import jax
import jax.numpy as jnp
from jax.experimental import pallas as pl


def kernel(target_table, context_table, pos_u, pos_v, neg_v):
    raise NotImplementedError("write your pallas kernel here")


# speedup vs baseline: 2.8122x; 2.8122x over previous
import jax
import jax.numpy as jnp
from jax import lax
from jax.experimental import pallas as pl
from jax.experimental.pallas import tpu as pltpu
from jax.experimental.pallas import tpu_sc as plsc

VOCAB = 100000
DIM = 64
B = 16384
K = 5

L = 16
NC, NS = 2, 16
NW = NC * NS
S = B // NW
C = 128
NCHUNK = S // C
QD = DIM // L


def _sc_scores(tgt, ctx, posu, posv, negf, pos_out, neg_out,
               posu_idx, posv_idx, neg_idx, t_rows, cp_rows, neg_rows,
               posd_v, negd_v, sem):
    wid = lax.axis_index("s") * NC + lax.axis_index("c")

    def chunk_body(c, carry):
        base = wid * S + c * C
        pltpu.sync_copy(posu.at[pl.ds(base, C)], posu_idx)
        pltpu.sync_copy(posv.at[pl.ds(base, C)], posv_idx)
        pltpu.sync_copy(negf.at[pl.ds(base * K, C * K)], neg_idx)
        cps = [pltpu.async_copy(tgt.at[posu_idx], t_rows, sem),
               pltpu.async_copy(ctx.at[posv_idx], cp_rows, sem)]
        for j in range(K):
            cps.append(pltpu.async_copy(ctx.at[neg_idx.at[pl.ds(j * C, C)]],
                                        neg_rows.at[pl.ds(j * C, C)], sem))
        for h in cps:
            h.wait()

        def item_body(i, icarry):
            t = [t_rows[i, pl.ds(q * L, L)] for q in range(QD)]
            cp = [cp_rows[i, pl.ds(q * L, L)] for q in range(QD)]
            accp = t[0] * cp[0]
            for q in range(1, QD):
                accp = accp + t[q] * cp[q]
            posd_v[pl.ds(i * L, L)] = accp
            cn = [neg_rows[i * K, pl.ds(q * L, L)] for q in range(QD)]
            for k in range(1, K):
                for q in range(QD):
                    cn[q] = cn[q] + neg_rows[i * K + k, pl.ds(q * L, L)]
            accn = t[0] * cn[0]
            for q in range(1, QD):
                accn = accn + t[q] * cn[q]
            negd_v[pl.ds(i * L, L)] = accn
            return icarry

        lax.fori_loop(0, C, item_body, 0)
        pltpu.sync_copy(posd_v, pos_out.at[pl.ds(base * L, C * L)])
        pltpu.sync_copy(negd_v, neg_out.at[pl.ds(base * L, C * L)])
        return carry

    lax.fori_loop(0, NCHUNK, chunk_body, 0)


def _tc_loss(pos_ref, neg_ref, out_ref):
    iu = lax.broadcasted_iota(jnp.int32, (128, 128 // L), 0)
    iv = lax.broadcasted_iota(jnp.int32, (128, 128 // L), 1)
    gsum = jnp.where(iu // L == iv, 1.0, 0.0).astype(jnp.float32)
    yp = jnp.dot(pos_ref[...], gsum, preferred_element_type=jnp.float32)
    yn = jnp.dot(neg_ref[...], gsum, preferred_element_type=jnp.float32)
    s = (jnp.sum(jnp.log(jax.nn.sigmoid(yp)))
         + jnp.sum(jnp.log(jax.nn.sigmoid(-yn))))
    out_ref[0, 0] = -s / jnp.float32(B + B * K)


def kernel(target_table, context_table, pos_u, pos_v, neg_v):
    negf = neg_v.reshape(B * K).astype(jnp.int32)
    pos_u = pos_u.astype(jnp.int32)
    pos_v = pos_v.astype(jnp.int32)

    mesh = plsc.VectorSubcoreMesh(core_axis_name="c", subcore_axis_name="s")
    sc_call = pl.kernel(
        _sc_scores, mesh=mesh,
        compiler_params=pltpu.CompilerParams(use_tc_tiling_on_sc=False),
        out_type=(jax.ShapeDtypeStruct((B * L,), jnp.float32),
                  jax.ShapeDtypeStruct((B * L,), jnp.float32)),
        scratch_types=[
            pltpu.VMEM((C,), jnp.int32),
            pltpu.VMEM((C,), jnp.int32),
            pltpu.VMEM((K * C,), jnp.int32),
            pltpu.VMEM((C, DIM), jnp.float32),
            pltpu.VMEM((C, DIM), jnp.float32),
            pltpu.VMEM((K * C, DIM), jnp.float32),
            pltpu.VMEM((C * L,), jnp.float32),
            pltpu.VMEM((C * L,), jnp.float32),
            pltpu.SemaphoreType.DMA,
        ],
    )
    pos_a, neg_a = sc_call(target_table, context_table, pos_u, pos_v, negf)

    out = pl.pallas_call(
        _tc_loss,
        out_shape=jax.ShapeDtypeStruct((1, 1), jnp.float32),
        out_specs=pl.BlockSpec(memory_space=pltpu.SMEM),
    )(pos_a.reshape(B * L // 128, 128), neg_a.reshape(B * L // 128, 128))
    return out[0, 0]
